# Initial kernel scaffold; baseline (speedup 1.0000x reference)
import jax
import jax.numpy as jnp
from jax.experimental import pallas as pl


def kernel(x, label, negs, table):
    raise NotImplementedError("write your pallas kernel here")


# speedup vs baseline: 1.3709x; 1.3709x over previous
import functools

import jax
import jax.numpy as jnp
from jax import lax
from jax.experimental import pallas as pl
from jax.experimental.pallas import tpu as pltpu
from jax.experimental.pallas import tpu_sc as plsc

_B = 128
_E = 128
_W = 5
_NNEG = 5
_NW = 32
_BPW = _B // _NW
_RPW = _BPW * (1 + _W + _NNEG)
_RPAD = 48

_mesh = plsc.VectorSubcoreMesh(core_axis_name="c", subcore_axis_name="s")


@functools.partial(
    pl.kernel,
    mesh=_mesh,
    out_type=(
        jax.ShapeDtypeStruct((_NW, _BPW, _E), jnp.float32),
        jax.ShapeDtypeStruct((_NW, _BPW, _E), jnp.float32),
        jax.ShapeDtypeStruct((_NW, _BPW, _E), jnp.float32),
    ),
    scratch_types=[
        pltpu.VMEM((_RPAD,), jnp.int32),
        pltpu.VMEM((_RPAD, _E), jnp.float32),
        pltpu.VMEM((_BPW, _E), jnp.float32),
        pltpu.VMEM((_BPW, _E), jnp.float32),
        pltpu.SemaphoreType.DMA,
    ],
)
def _sc_gather_mean(idx_hbm, table_hbm, u_out, vp_out, vn_out,
                    idx_v, rows_v, vp_v, vn_v, sem):
    wid = lax.axis_index("s") * 2 + lax.axis_index("c")
    pltpu.sync_copy(idx_hbm.at[pl.ds(wid * _RPAD, _RPAD)], idx_v)
    pltpu.async_copy(table_hbm.at[idx_v], rows_v, sem).wait()
    inv = jnp.float32(1.0 / _W)
    for j in range(_BPW):
        for c in range(_E // 16):
            sl = pl.ds(c * 16, 16)
            vp = rows_v[_BPW + _W * j, sl]
            for t in range(1, _W):
                vp = vp + rows_v[_BPW + _W * j + t, sl]
            vp_v[j, sl] = vp * inv
            base_n = _BPW + _BPW * _W
            vn = rows_v[base_n + _NNEG * j, sl]
            for t in range(1, _NNEG):
                vn = vn + rows_v[base_n + _NNEG * j + t, sl]
            vn_v[j, sl] = vn * inv
    pltpu.sync_copy(rows_v.at[pl.ds(0, _BPW)], u_out.at[wid])
    pltpu.sync_copy(vp_v, vp_out.at[wid])
    pltpu.sync_copy(vn_v, vn_out.at[wid])


def _tc_loss_body(u_ref, vp_ref, vn_ref, out_ref):
    u = u_ref[...]
    vp = vp_ref[...]
    vn = vn_ref[...]
    m = jnp.dot(u, vp, preferred_element_type=jnp.float32)
    ri = lax.broadcasted_iota(jnp.int32, (_B, _B), 0)
    ci = lax.broadcasted_iota(jnp.int32, (_B, _B), 1)
    s1 = jnp.sum(jnp.where(ri == ci, m, 0.0), axis=1, keepdims=True)
    s2 = jnp.sum(u * vn, axis=1, keepdims=True)
    l1 = -jnp.log(1.0 / (1.0 + jnp.exp(-s1)))
    l2 = -jnp.log(1.0 / (1.0 + jnp.exp(s2)))
    out_ref[...] = jnp.reshape(jnp.mean(l1) + jnp.mean(l2), (1, 1))


_tc_loss = pl.pallas_call(
    _tc_loss_body,
    out_shape=jax.ShapeDtypeStruct((1, 1), jnp.float32),
)


def kernel(x, label, negs, table):
    lab = label.reshape(_NW, _BPW)
    xr = x.reshape(_NW, _BPW * _W)
    nr = negs.reshape(_NW, _BPW * _NNEG)
    pad = jnp.zeros((_NW, _RPAD - _RPW), jnp.int32)
    idx = jnp.concatenate([lab, xr, nr, pad], axis=1).reshape(-1)
    u3, vp3, vn3 = _sc_gather_mean(idx, table)
    loss = _tc_loss(u3.reshape(_B, _E), vp3.reshape(_B, _E),
                    vn3.reshape(_B, _E))
    return loss[0, 0]


# speedup vs baseline: 1.3753x; 1.0032x over previous
import functools

import jax
import jax.numpy as jnp
from jax import lax
from jax.experimental import pallas as pl
from jax.experimental.pallas import tpu as pltpu
from jax.experimental.pallas import tpu_sc as plsc

_B = 128
_E = 128
_W = 5
_NNEG = 5
_NW = 32
_BPW = _B // _NW
_RPW = _BPW * (1 + _W + _NNEG)
_RPAD = 48

_mesh = plsc.VectorSubcoreMesh(core_axis_name="c", subcore_axis_name="s")


@functools.partial(
    pl.kernel,
    mesh=_mesh,
    out_type=(
        jax.ShapeDtypeStruct((_B, _E), jnp.float32),
        jax.ShapeDtypeStruct((_B, _E), jnp.float32),
        jax.ShapeDtypeStruct((_B, _E), jnp.float32),
    ),
    scratch_types=[
        pltpu.VMEM((_RPAD,), jnp.int32),
        pltpu.VMEM((_RPAD, _E), jnp.float32),
        pltpu.VMEM((_BPW, _E), jnp.float32),
        pltpu.VMEM((_BPW, _E), jnp.float32),
        pltpu.SemaphoreType.DMA,
    ],
)
def _sc_gather_mean(idx_hbm, table_hbm, u_out, vp_out, vn_out,
                    idx_v, rows_v, vp_v, vn_v, sem):
    wid = lax.axis_index("s") * 2 + lax.axis_index("c")
    pltpu.sync_copy(idx_hbm.at[pl.ds(wid * _RPAD, _RPAD)], idx_v)
    pltpu.async_copy(table_hbm.at[idx_v], rows_v, sem).wait()
    inv = jnp.float32(1.0 / _W)
    for j in range(_BPW):
        for c in range(_E // 16):
            sl = pl.ds(c * 16, 16)
            vp = rows_v[_BPW + _W * j, sl]
            for t in range(1, _W):
                vp = vp + rows_v[_BPW + _W * j + t, sl]
            vp_v[j, sl] = vp * inv
            base_n = _BPW + _BPW * _W
            vn = rows_v[base_n + _NNEG * j, sl]
            for t in range(1, _NNEG):
                vn = vn + rows_v[base_n + _NNEG * j + t, sl]
            vn_v[j, sl] = vn * inv
    rsl = pl.ds(wid * _BPW, _BPW)
    pltpu.sync_copy(rows_v.at[pl.ds(0, _BPW)], u_out.at[rsl])
    pltpu.sync_copy(vp_v, vp_out.at[rsl])
    pltpu.sync_copy(vn_v, vn_out.at[rsl])


def _tc_loss_body(u_ref, vp_ref, vn_ref, out_ref):
    u = u_ref[...]
    vp = vp_ref[...]
    vn = vn_ref[...]
    m = jnp.dot(u, vp, preferred_element_type=jnp.float32)
    ri = lax.broadcasted_iota(jnp.int32, (_B, _B), 0)
    ci = lax.broadcasted_iota(jnp.int32, (_B, _B), 1)
    s1 = jnp.sum(jnp.where(ri == ci, m, 0.0), axis=1, keepdims=True)
    s2 = jnp.sum(u * vn, axis=1, keepdims=True)
    l1 = -jnp.log(1.0 / (1.0 + jnp.exp(-s1)))
    l2 = -jnp.log(1.0 / (1.0 + jnp.exp(s2)))
    out_ref[...] = jnp.reshape(jnp.mean(l1) + jnp.mean(l2), (1, 1))


_tc_loss = pl.pallas_call(
    _tc_loss_body,
    out_shape=jax.ShapeDtypeStruct((1, 1), jnp.float32),
)


def kernel(x, label, negs, table):
    lab = label.reshape(_NW, _BPW)
    xr = x.reshape(_NW, _BPW * _W)
    nr = negs.reshape(_NW, _BPW * _NNEG)
    pad = jnp.zeros((_NW, _RPAD - _RPW), jnp.int32)
    idx = jnp.concatenate([lab, xr, nr, pad], axis=1).reshape(-1)
    u, vp, vn = _sc_gather_mean(idx, table)
    loss = _tc_loss(u, vp, vn)
    return loss[0, 0]


# speedup vs baseline: 1.6026x; 1.1652x over previous
import functools

import jax
import jax.numpy as jnp
from jax import lax
from jax.experimental import pallas as pl
from jax.experimental.pallas import tpu as pltpu
from jax.experimental.pallas import tpu_sc as plsc

_B = 128
_E = 128
_W = 5
_NNEG = 5
_NWK = 16
_BPW = _B // _NWK
_RPW = _BPW * (1 + _W + _NNEG)
_NL = 16
_NCH = _E // _NL

_mesh = plsc.VectorSubcoreMesh(
    core_axis_name="c", subcore_axis_name="s", num_cores=1)


def _allsum16(v):
    iota = lax.iota(jnp.int32, _NL)
    for sh in (1, 2, 4, 8):
        v = v + v.at[jnp.bitwise_xor(iota, sh)].get(mode="promise_in_bounds")
    return v


def _softplus16(t):
    u = jnp.exp(-jnp.abs(t))
    z = u / (u + 2.0)
    z2 = z * z
    ln1p = 2.0 * z * (1.0 + z2 * (1.0 / 3.0 + z2 * (0.2 + z2 * (1.0 / 7.0))))
    return jnp.maximum(t, 0.0) + ln1p


@functools.partial(
    pl.kernel,
    mesh=_mesh,
    out_type=jax.ShapeDtypeStruct((_NL,), jnp.float32),
    compiler_params=pltpu.CompilerParams(needs_layout_passes=False),
    scratch_types=[
        pltpu.VMEM((_RPW,), jnp.int32),
        pltpu.VMEM((_RPW, _E), jnp.float32),
        pltpu.VMEM((_BPW * _E,), jnp.float32),
        pltpu.VMEM((_B * _E,), jnp.float32),
        pltpu.VMEM((_NL,), jnp.float32),
        pltpu.SMEM((1,), jnp.int32),
        pltpu.VMEM_SHARED((_B * _E,), jnp.float32),
        pltpu.SemaphoreType.DMA,
    ],
)
def _sc_loss(idx_hbm, table_hbm, out_hbm,
             idx_v, rows_v, vp_v, vp_all, loss_v, acc_smem, vp_sh, sem):
    wid = lax.axis_index("s")
    acc_smem[0] = 0
    pltpu.sync_copy(idx_hbm.at[pl.ds(wid * _RPW, _RPW)], idx_v)
    pltpu.async_copy(table_hbm.at[idx_v], rows_v, sem).wait()

    inv = jnp.float32(1.0 / _W)
    iota = lax.iota(jnp.int32, _NL)
    fzero = jnp.zeros((_NL,), jnp.float32)

    s2 = [None] * _BPW
    for j in range(_BPW):
        acc2 = fzero
        for c in range(_NCH):
            sl = pl.ds(c * _NL, _NL)
            vp = rows_v[_BPW + _W * j, sl]
            for t in range(1, _W):
                vp = vp + rows_v[_BPW + _W * j + t, sl]
            vp_v[pl.ds(j * _E + c * _NL, _NL)] = vp * inv
            base_n = _BPW * (1 + _W)
            vn = rows_v[base_n + _NNEG * j, sl]
            for t in range(1, _NNEG):
                vn = vn + rows_v[base_n + _NNEG * j + t, sl]
            acc2 = acc2 + rows_v[j, sl] * (vn * inv)
        s2[j] = _allsum16(acc2)

    pltpu.sync_copy(vp_v, vp_sh.at[pl.ds(wid * _BPW * _E, _BPW * _E)])
    plsc.subcore_barrier()
    pltpu.sync_copy(vp_sh, vp_all)

    s1 = [None] * _BPW
    for j in range(_BPW):
        i = wid * _BPW + j
        col = jnp.full((_NL,), i, jnp.int32)
        acc1 = fzero
        for c in range(_NCH):
            flat = (iota + (c * _NL)) * _E + col
            g = plsc.load_gather(vp_all, [flat])
            acc1 = acc1 + rows_v[j, pl.ds(c * _NL, _NL)] * g
        s1[j] = _allsum16(acc1)

    v = fzero
    for j in range(_BPW):
        v = jnp.where(iota == j, -s1[j], v)
        v = jnp.where(iota == (_BPW + j), s2[j], v)
    sp = _softplus16(v)

    psum = _allsum16(sp)
    pi = ((psum * jnp.float32(1048576.0))
          + jnp.float32(0.5)).astype(jnp.int32)
    plsc.subcore_barrier()
    plsc.fetch_and_add(acc_smem.at[0], pi[0], subcore_id=0)
    plsc.subcore_barrier()

    @pl.when(wid == 0)
    def _():
        tot = acc_smem[0]
        loss = tot.astype(jnp.float32) * jnp.float32(1.0 / (1048576.0 * _B))
        loss_v[...] = jnp.full((_NL,), loss, jnp.float32)
        pltpu.sync_copy(loss_v, out_hbm)


def kernel(x, label, negs, table):
    lab = label.reshape(_NWK, _BPW)
    xr = x.reshape(_NWK, _BPW * _W)
    nr = negs.reshape(_NWK, _BPW * _NNEG)
    idx = jnp.concatenate([lab, xr, nr], axis=1).reshape(-1)
    out = _sc_loss(idx, table)
    return out[0]
